# baseline (speedup 1.0000x reference)
import dataclasses
import functools

import jax
import jax.numpy as jnp
from jax import lax
from jax.experimental import pallas as pl
from jax.experimental.pallas import tpu as pltpu
from jax.experimental.pallas import tpu_sc as plsc

N = 10000
D = 128
H = 128
E = 320000

NC = 2
NS = 16
NW = NC * NS
CH = 128

HBUF = 8

EPT = -(-E // (NW * CH * HBUF)) * CH * HBUF
EP = EPT * NW
NP = 10112
RPS = NP // NS
NCHUNK = EPT // CH

_mesh = plsc.VectorSubcoreMesh(core_axis_name="c", subcore_axis_name="s")

_cp = pltpu.CompilerParams()
if "needs_layout_passes" in pltpu.CompilerParams.__dataclass_fields__:
  _cp = dataclasses.replace(_cp, needs_layout_passes=False)


def _sc_hist(dstp2, zrows, ones):
  @functools.partial(
      pl.kernel,
      out_type=jax.ShapeDtypeStruct((NC, NP, H), jnp.float32),
      mesh=_mesh,
      scratch_types=[
          pltpu.VMEM((NCHUNK, CH), jnp.int32),
          pltpu.VMEM((CH, H), jnp.float32),
          pltpu.VMEM_SHARED((NP, H), jnp.float32),
          pltpu.SemaphoreType.DMA((HBUF,)),
          pltpu.SemaphoreType.DMA,
      ],
  )
  def k(dst_hbm, z_hbm, ones_hbm, out_hbm, di_v, ones_v, acc, ssem, isem):
    c = lax.axis_index("c")
    s = lax.axis_index("s")
    wid = s * NC + c
    pltpu.async_copy(dst_hbm.at[pl.ds(wid * NCHUNK, NCHUNK)], di_v, isem)
    pltpu.async_copy(ones_hbm, ones_v, isem)
    pltpu.async_copy(z_hbm, acc.at[pl.ds(s * RPS, RPS)], isem)
    pltpu.make_async_copy(dst_hbm.at[pl.ds(wid * NCHUNK, NCHUNK)], di_v,
                          isem).wait()
    pltpu.make_async_copy(ones_hbm, ones_v, isem).wait()
    pltpu.make_async_copy(z_hbm, acc.at[pl.ds(s * RPS, RPS)], isem).wait()
    plsc.subcore_barrier()

    @pl.loop(0, NCHUNK, step=HBUF)
    def _(j):
      for b in range(HBUF):
        ch = j + b

        @pl.when(j > 0)
        def _():
          pltpu.make_async_copy(ones_v, acc.at[di_v.at[ch - HBUF]],
                                ssem.at[b]).wait()

        pltpu.make_async_copy(ones_v, acc.at[di_v.at[ch]],
                              ssem.at[b]).start(add=True)

    for b in range(HBUF):
      pltpu.make_async_copy(ones_v, acc.at[di_v.at[NCHUNK - HBUF + b]],
                            ssem.at[b]).wait()

    plsc.subcore_barrier()
    pltpu.sync_copy(acc.at[pl.ds(s * RPS, RPS)],
                    out_hbm.at[c, pl.ds(s * RPS, RPS)])

  return k(dstp2, zrows, ones)


def _sc_agg(y, srcp, dstp, zrows):
  @functools.partial(
      pl.kernel,
      out_type=jax.ShapeDtypeStruct((NC, NP, H), jnp.float32),
      mesh=_mesh,
      scratch_types=[
          pltpu.VMEM((CH,), jnp.int32),
          pltpu.VMEM((CH,), jnp.int32),
          pltpu.VMEM((CH, H), jnp.float32),
          pltpu.VMEM_SHARED((NP, H), jnp.float32),
      ],
  )
  def k(y_hbm, src_hbm, dst_hbm, z_hbm, out_hbm, si_v, di_v, rows_v, acc):
    c = lax.axis_index("c")
    s = lax.axis_index("s")
    wid = s * NC + c
    pltpu.sync_copy(z_hbm, acc.at[pl.ds(s * RPS, RPS)])
    plsc.subcore_barrier()

    @pl.loop(0, NCHUNK)
    def _(i):
      base = wid * EPT + i * CH
      pltpu.sync_copy(src_hbm.at[pl.ds(base, CH)], si_v)
      pltpu.sync_copy(dst_hbm.at[pl.ds(base, CH)], di_v)
      pltpu.sync_copy(y_hbm.at[si_v], rows_v)
      pltpu.sync_copy(rows_v, acc.at[di_v], add=True)

    plsc.subcore_barrier()
    pltpu.sync_copy(acc.at[pl.ds(s * RPS, RPS)],
                    out_hbm.at[c, pl.ds(s * RPS, RPS)])

  return k(y, srcp, dstp, zrows)


BR = 2000
NR = NP // 128


def _tc_first(hist, x, W1):
  def body(h_ref, x_ref, w_ref, y_ref):
    dinv = lax.rsqrt(h_ref[0] + h_ref[1] + 1.0)
    xw = jnp.dot(x_ref[...], w_ref[...], preferred_element_type=jnp.float32)
    y_ref[...] = dinv * xw

  return pl.pallas_call(
      body,
      grid=(N // BR,),
      in_specs=[pl.BlockSpec((NC, BR, H), lambda i: (0, i, 0)),
                pl.BlockSpec((BR, D), lambda i: (i, 0)),
                pl.BlockSpec((D, H), lambda i: (0, 0))],
      out_specs=pl.BlockSpec((BR, H), lambda i: (i, 0)),
      out_shape=jax.ShapeDtypeStruct((N, H), jnp.float32),
  )(hist, x, W1)


def _tc_fuse_mid(hist, agg, y1, b1, W2):
  def body(h_ref, a_ref, y_ref, b_ref, w_ref, o_ref):
    dinv = lax.rsqrt(h_ref[0] + h_ref[1] + 1.0)
    hmid = jnp.maximum(
        dinv * (a_ref[0] + a_ref[1] + y_ref[...]) + b_ref[...], 0.0)
    o_ref[...] = dinv * jnp.dot(hmid, w_ref[...],
                                preferred_element_type=jnp.float32)

  return pl.pallas_call(
      body,
      grid=(N // BR,),
      in_specs=[pl.BlockSpec((NC, BR, H), lambda i: (0, i, 0)),
                pl.BlockSpec((NC, BR, H), lambda i: (0, i, 0)),
                pl.BlockSpec((BR, H), lambda i: (i, 0)),
                pl.BlockSpec((1, H), lambda i: (0, 0)),
                pl.BlockSpec((D, H), lambda i: (0, 0))],
      out_specs=pl.BlockSpec((BR, H), lambda i: (i, 0)),
      out_shape=jax.ShapeDtypeStruct((N, H), jnp.float32),
  )(hist, agg, y1, b1, W2)


def _tc_fuse_out(hist, agg, y2, b2):
  def body(h_ref, a_ref, y_ref, b_ref, o_ref):
    dinv = lax.rsqrt(h_ref[0] + h_ref[1] + 1.0)
    o_ref[...] = dinv * (a_ref[0] + a_ref[1] + y_ref[...]) + b_ref[...]

  return pl.pallas_call(
      body,
      grid=(N // BR,),
      in_specs=[pl.BlockSpec((NC, BR, H), lambda i: (0, i, 0)),
                pl.BlockSpec((NC, BR, H), lambda i: (0, i, 0)),
                pl.BlockSpec((BR, H), lambda i: (i, 0)),
                pl.BlockSpec((1, H), lambda i: (0, 0))],
      out_specs=pl.BlockSpec((BR, H), lambda i: (i, 0)),
      out_shape=jax.ShapeDtypeStruct((N, H), jnp.float32),
  )(hist, agg, y2, b2)


def kernel(x, edge_index, W1, b1, W2, b2):
  src = edge_index[0]
  dst = edge_index[1]
  pad = EP - E
  srcp = jnp.concatenate([src, jnp.zeros((pad,), jnp.int32)])
  dstp = jnp.concatenate([dst, jnp.full((pad,), N, jnp.int32)])
  dstp2 = dstp.reshape(NW * NCHUNK, CH)
  zrows = jnp.zeros((RPS, H), jnp.float32)
  ones = jnp.ones((CH, H), jnp.float32)
  b1r = b1.reshape(1, H)
  b2r = b2.reshape(1, H)

  hist = _sc_hist(dstp2, zrows, ones)
  y1 = _tc_first(hist, x, W1)
  a1 = _sc_agg(y1, srcp, dstp, zrows)
  y2 = _tc_fuse_mid(hist, a1, y1, b1r, W2)
  a2 = _sc_agg(y2, srcp, dstp, zrows)
  out = _tc_fuse_out(hist, a2, y2, b2r)
  return out


# speedup vs baseline: 1.4964x; 1.4964x over previous
import functools

import jax
import jax.numpy as jnp
from jax import lax
from jax.experimental import pallas as pl
from jax.experimental.pallas import tpu as pltpu
from jax.experimental.pallas import tpu_sc as plsc

N = 10000
D = 128
H = 128
E = 320000

NC = 2
NS = 16
NW = NC * NS
CH = 128

EPT = ((E + NW * CH - 1) // (NW * CH)) * CH
EP = EPT * NW
NP = 10112
RPS = NP // NS
NCHUNK = EPT // CH

_mesh = plsc.VectorSubcoreMesh(core_axis_name="c", subcore_axis_name="s")


def _sc_hist(dstp, zrows, ones):
  @functools.partial(
      pl.kernel,
      out_type=jax.ShapeDtypeStruct((NC, NP, H), jnp.float32),
      mesh=_mesh,
      scratch_types=[
          pltpu.VMEM((CH,), jnp.int32),
          pltpu.VMEM((CH, H), jnp.float32),
          pltpu.VMEM_SHARED((NP, H), jnp.float32),
      ],
  )
  def k(dst_hbm, z_hbm, ones_hbm, out_hbm, idx_v, ones_v, acc):
    c = lax.axis_index("c")
    s = lax.axis_index("s")
    wid = s * NC + c
    pltpu.sync_copy(ones_hbm, ones_v)
    pltpu.sync_copy(z_hbm, acc.at[pl.ds(s * RPS, RPS)])
    plsc.subcore_barrier()

    @pl.loop(0, NCHUNK)
    def _(i):
      base = wid * EPT + i * CH
      pltpu.sync_copy(dst_hbm.at[pl.ds(base, CH)], idx_v)
      pltpu.sync_copy(ones_v, acc.at[idx_v], add=True)

    plsc.subcore_barrier()
    pltpu.sync_copy(acc.at[pl.ds(s * RPS, RPS)],
                    out_hbm.at[c, pl.ds(s * RPS, RPS)])

  return k(dstp, zrows, ones)


def _sc_agg(y, srcp, dstp, zrows):
  @functools.partial(
      pl.kernel,
      out_type=jax.ShapeDtypeStruct((NC, NP, H), jnp.float32),
      mesh=_mesh,
      scratch_types=[
          pltpu.VMEM((CH,), jnp.int32),
          pltpu.VMEM((CH,), jnp.int32),
          pltpu.VMEM((CH, H), jnp.float32),
          pltpu.VMEM_SHARED((NP, H), jnp.float32),
      ],
  )
  def k(y_hbm, src_hbm, dst_hbm, z_hbm, out_hbm, si_v, di_v, rows_v, acc):
    c = lax.axis_index("c")
    s = lax.axis_index("s")
    wid = s * NC + c
    pltpu.sync_copy(z_hbm, acc.at[pl.ds(s * RPS, RPS)])
    plsc.subcore_barrier()

    @pl.loop(0, NCHUNK)
    def _(i):
      base = wid * EPT + i * CH
      pltpu.sync_copy(src_hbm.at[pl.ds(base, CH)], si_v)
      pltpu.sync_copy(dst_hbm.at[pl.ds(base, CH)], di_v)
      pltpu.sync_copy(y_hbm.at[si_v], rows_v)
      pltpu.sync_copy(rows_v, acc.at[di_v], add=True)

    plsc.subcore_barrier()
    pltpu.sync_copy(acc.at[pl.ds(s * RPS, RPS)],
                    out_hbm.at[c, pl.ds(s * RPS, RPS)])

  return k(y, srcp, dstp, zrows)


BR = 2000


def _tc_matmul(x, W):
  def body(x_ref, w_ref, o_ref):
    o_ref[...] = jnp.dot(x_ref[...], w_ref[...],
                         preferred_element_type=jnp.float32)

  return pl.pallas_call(
      body,
      grid=(N // BR,),
      in_specs=[pl.BlockSpec((BR, D), lambda i: (i, 0)),
                pl.BlockSpec((D, H), lambda i: (0, 0))],
      out_specs=pl.BlockSpec((BR, H), lambda i: (i, 0)),
      out_shape=jax.ShapeDtypeStruct((N, H), jnp.float32),
  )(x, W)


def _tc_scale(hist, xw):
  def body(h_ref, x_ref, o_ref):
    dinv = lax.rsqrt(h_ref[0] + h_ref[1] + 1.0)
    o_ref[...] = dinv * x_ref[...]

  return pl.pallas_call(
      body,
      grid=(N // BR,),
      in_specs=[pl.BlockSpec((NC, BR, H), lambda i: (0, i, 0)),
                pl.BlockSpec((BR, H), lambda i: (i, 0))],
      out_specs=pl.BlockSpec((BR, H), lambda i: (i, 0)),
      out_shape=jax.ShapeDtypeStruct((N, H), jnp.float32),
  )(hist, xw)


def _tc_fuse_mid(hist, agg, y1, b1, W2):
  def body(h_ref, a_ref, y_ref, b_ref, w_ref, o_ref):
    dinv = lax.rsqrt(h_ref[0] + h_ref[1] + 1.0)
    hmid = jnp.maximum(
        dinv * (a_ref[0] + a_ref[1] + y_ref[...]) + b_ref[...], 0.0)
    o_ref[...] = dinv * jnp.dot(hmid, w_ref[...],
                                preferred_element_type=jnp.float32)

  return pl.pallas_call(
      body,
      grid=(N // BR,),
      in_specs=[pl.BlockSpec((NC, BR, H), lambda i: (0, i, 0)),
                pl.BlockSpec((NC, BR, H), lambda i: (0, i, 0)),
                pl.BlockSpec((BR, H), lambda i: (i, 0)),
                pl.BlockSpec((1, H), lambda i: (0, 0)),
                pl.BlockSpec((D, H), lambda i: (0, 0))],
      out_specs=pl.BlockSpec((BR, H), lambda i: (i, 0)),
      out_shape=jax.ShapeDtypeStruct((N, H), jnp.float32),
  )(hist, agg, y1, b1, W2)


def _tc_fuse_out(hist, agg, y2, b2):
  def body(h_ref, a_ref, y_ref, b_ref, o_ref):
    dinv = lax.rsqrt(h_ref[0] + h_ref[1] + 1.0)
    o_ref[...] = dinv * (a_ref[0] + a_ref[1] + y_ref[...]) + b_ref[...]

  return pl.pallas_call(
      body,
      grid=(N // BR,),
      in_specs=[pl.BlockSpec((NC, BR, H), lambda i: (0, i, 0)),
                pl.BlockSpec((NC, BR, H), lambda i: (0, i, 0)),
                pl.BlockSpec((BR, H), lambda i: (i, 0)),
                pl.BlockSpec((1, H), lambda i: (0, 0))],
      out_specs=pl.BlockSpec((BR, H), lambda i: (i, 0)),
      out_shape=jax.ShapeDtypeStruct((N, H), jnp.float32),
  )(hist, agg, y2, b2)


def kernel(x, edge_index, W1, b1, W2, b2):
  src = edge_index[0]
  dst = edge_index[1]
  ppt = EPT - E // NW
  pad_dst = jnp.broadcast_to(N + jnp.arange(ppt, dtype=jnp.int32),
                             (NW, ppt))
  pad_src = jnp.zeros((NW, ppt), jnp.int32)
  srcp = jnp.concatenate([src.reshape(NW, E // NW), pad_src],
                         axis=1).reshape(EP)
  dstp = jnp.concatenate([dst.reshape(NW, E // NW), pad_dst],
                         axis=1).reshape(EP)
  zrows = jnp.zeros((RPS, H), jnp.float32)
  ones = jnp.ones((CH, H), jnp.float32)
  b1r = b1.reshape(1, H)
  b2r = b2.reshape(1, H)

  hist = _sc_hist(dstp, zrows, ones)
  xw1 = _tc_matmul(x, W1)
  y1 = _tc_scale(hist, xw1)
  a1 = _sc_agg(y1, srcp, dstp, zrows)
  y2 = _tc_fuse_mid(hist, a1, y1, b1r, W2)
  a2 = _sc_agg(y2, srcp, dstp, zrows)
  out = _tc_fuse_out(hist, a2, y2, b2r)
  return out
